# baseline (speedup 1.0000x reference)
import functools

import jax
import jax.numpy as jnp
from jax import lax
from jax.experimental import pallas as pl
from jax.experimental.pallas import tpu as pltpu
from jax.experimental.pallas import tpu_sc as plsc

N = 10000
E = 320000
D = 128
NUM_RELS = 200

NC = 2
NS = 16
NW = NC * NS
CHUNK = 56
NCHUNK = 192
EPW = NCHUNK * CHUNK
IDXC = 24
NSUP = NCHUNK // IDXC
GRP = 6
NGRP = IDXC // GRP
EPAD = EPW * NW
NPAD = 10112
ROWS_PT = NPAD // NS
LANES = 16


def _sc_message_accumulate(src3, dst3, et3, x, re_ext, zrows):
  mesh = plsc.VectorSubcoreMesh(core_axis_name="c", subcore_axis_name="s")

  @functools.partial(
      pl.kernel,
      out_type=jax.ShapeDtypeStruct((NC, NPAD, D), jnp.float32),
      mesh=mesh,
      scratch_types=[
          pltpu.VMEM((IDXC, CHUNK), jnp.int32),
          pltpu.VMEM((IDXC, CHUNK), jnp.int32),
          pltpu.VMEM((IDXC, CHUNK), jnp.int32),
          pltpu.VMEM((CHUNK, D), jnp.float32),
          pltpu.VMEM((CHUNK, D), jnp.float32),
          pltpu.VMEM((CHUNK, D), jnp.float32),
          pltpu.VMEM((CHUNK, D), jnp.float32),
          pltpu.VMEM((CHUNK, D), jnp.float32),
          pltpu.VMEM_SHARED((NPAD, D), jnp.float32),
          pltpu.SemaphoreType.DMA,
          pltpu.SemaphoreType.DMA,
          pltpu.SemaphoreType.DMA,
          pltpu.SemaphoreType.DMA,
      ],
  )
  def k(src_hbm, dst_hbm, et_hbm, x_hbm, re_hbm, z_hbm, out_hbm,
        src_v, dst_v, et_v, xr0, xr1, xr2, rr0, rr1, acc_sh,
        sem_g0, sem_g1, sem_s0, sem_s1):
    cid = lax.axis_index("c")
    sid = lax.axis_index("s")
    wid = cid * NS + sid
    xr = (xr0, xr1, xr2)
    rr = (rr0, rr1)
    sem_g = (sem_g0, sem_g1)
    sem_s = (sem_s0, sem_s1)

    pltpu.sync_copy(z_hbm, acc_sh.at[pl.ds(sid * ROWS_PT, ROWS_PT)])
    plsc.subcore_barrier()

    def issue_gather(c, k):
      pltpu.async_copy(x_hbm.at[src_v.at[c]], xr[k % 3], sem_g[k % 2])
      pltpu.async_copy(re_hbm.at[et_v.at[c]], rr[k % 2], sem_g[k % 2])

    def wait_gather(c, k):
      pltpu.make_async_copy(x_hbm.at[src_v.at[c]], xr[k % 3],
                            sem_g[k % 2]).wait()
      pltpu.make_async_copy(re_hbm.at[et_v.at[c]], rr[k % 2],
                            sem_g[k % 2]).wait()

    def issue_scatter(c, k):
      pltpu.async_copy(xr[k % 3], acc_sh.at[dst_v.at[c]], sem_s[k % 2],
                       add=True)

    def wait_scatter(c, k):
      pltpu.make_async_copy(xr[k % 3], acc_sh.at[dst_v.at[c]],
                            sem_s[k % 2]).wait()

    def mul(k):
      xk, rk = xr[k % 3], rr[k % 2]

      def mul_body(e, carry2):
        for j in range(D // LANES):
          s = pl.ds(j * LANES, LANES)
          xk[e, s] = xk[e, s] * rk[e, s]
        return carry2

      lax.fori_loop(0, CHUNK, mul_body, 0)

    def super_body(sc, carry):
      pltpu.sync_copy(src_hbm.at[wid, sc], src_v)
      pltpu.sync_copy(dst_hbm.at[wid, sc], dst_v)
      pltpu.sync_copy(et_hbm.at[wid, sc], et_v)

      issue_gather(0, 0)

      def group_body(g, carry1):
        base = g * GRP
        for k in range(GRP):
          c = base + k
          if k >= 2:
            wait_scatter(c - 2, k - 2)
          else:
            @pl.when(g >= 1)
            def _():
              wait_scatter(c - 2, k + GRP - 2)
          if k < GRP - 1:
            issue_gather(c + 1, k + 1)
          else:
            @pl.when(g + 1 < NGRP)
            def _():
              issue_gather(c + 1, 0)
          wait_gather(c, k)
          mul(k)
          issue_scatter(c, k)
        return carry1

      lax.fori_loop(0, NGRP, group_body, 0)
      wait_scatter(IDXC - 2, GRP - 2)
      wait_scatter(IDXC - 1, GRP - 1)
      return carry

    lax.fori_loop(0, NSUP, super_body, 0)
    plsc.subcore_barrier()

    r0 = sid * ROWS_PT
    pltpu.sync_copy(acc_sh.at[pl.ds(r0, ROWS_PT)],
                    out_hbm.at[cid, pl.ds(r0, ROWS_PT)])

  return k(src3, dst3, et3, x, re_ext, zrows)


def _tc_out_matmul(partials, weight, bias2d):
  BM = 1264

  def body(p_ref, w_ref, b_ref, o_ref):
    acc = jnp.dot(p_ref[0] + p_ref[1], w_ref[...],
                  preferred_element_type=jnp.float32)
    o_ref[...] = acc + b_ref[...]

  return pl.pallas_call(
      body,
      grid=(NPAD // BM,),
      in_specs=[
          pl.BlockSpec((NC, BM, D), lambda i: (0, i, 0)),
          pl.BlockSpec((D, D), lambda i: (0, 0)),
          pl.BlockSpec((1, D), lambda i: (0, 0)),
      ],
      out_specs=pl.BlockSpec((BM, D), lambda i: (i, 0)),
      out_shape=jax.ShapeDtypeStruct((NPAD, D), jnp.float32),
  )(partials, weight, bias2d)


def _tc_rel_matmul(re_pad, rel_weight):
  def body(r_ref, w_ref, o_ref):
    o_ref[...] = jnp.dot(r_ref[...], w_ref[...],
                         preferred_element_type=jnp.float32)

  return pl.pallas_call(
      body,
      out_shape=jax.ShapeDtypeStruct((re_pad.shape[0], D), jnp.float32),
  )(re_pad, rel_weight)


def kernel(x, edge_index, edge_type, rel_embed, weight, rel_weight, bias):
  src = edge_index[0]
  dst = edge_index[1]
  npad = EPAD - E
  spread = jnp.arange(npad, dtype=jnp.int32) % N
  src3 = jnp.concatenate([src, spread]).reshape(NW, NSUP, IDXC, CHUNK)
  dst3 = jnp.concatenate([dst, spread]).reshape(NW, NSUP, IDXC, CHUNK)
  et3 = jnp.concatenate(
      [edge_type, jnp.full((npad,), 2 * NUM_RELS, jnp.int32)]
  ).reshape(NW, NSUP, IDXC, CHUNK)

  re_ext = jnp.concatenate(
      [rel_embed, jnp.zeros((1, D), rel_embed.dtype)], axis=0)
  zrows = jnp.zeros((ROWS_PT, D), jnp.float32)

  partials = _sc_message_accumulate(src3, dst3, et3, x, re_ext, zrows)
  out = _tc_out_matmul(partials, weight, bias.reshape(1, D))[:N]

  re_pad = jnp.concatenate(
      [re_ext, jnp.zeros((7, D), rel_embed.dtype)], axis=0)
  rel_out = _tc_rel_matmul(re_pad, rel_weight)[:2 * NUM_RELS + 1]
  return (out, rel_out)


# speedup vs baseline: 1.8436x; 1.8436x over previous
import functools

import jax
import jax.numpy as jnp
from jax import lax
from jax.experimental import pallas as pl
from jax.experimental.pallas import tpu as pltpu
from jax.experimental.pallas import tpu_sc as plsc

N = 10000
E = 320000
D = 128
NUM_RELS = 200

NC = 2
NS = 16
NW = NC * NS
CHUNK = 64
GIDX = 2 * CHUNK
NCHUNK = 160
EPW = NCHUNK * CHUNK
IDXC = 40
NSUP = NCHUNK // IDXC
NPAIR = IDXC // 2
EPAD = EPW * NW
NPAD = 10112
ROWS_PT = NPAD // NS
LANES = 16


def _sc_message_accumulate(gidx4, dst4, tab, zrows):
  mesh = plsc.VectorSubcoreMesh(core_axis_name="c", subcore_axis_name="s")

  @functools.partial(
      pl.kernel,
      out_type=jax.ShapeDtypeStruct((NC, NPAD, D), jnp.float32),
      mesh=mesh,
      scratch_types=[
          pltpu.VMEM((IDXC, GIDX), jnp.int32),
          pltpu.VMEM((IDXC, CHUNK), jnp.int32),
          pltpu.VMEM((GIDX, D), jnp.float32),
          pltpu.VMEM((GIDX, D), jnp.float32),
          pltpu.VMEM_SHARED((NPAD, D), jnp.float32),
          pltpu.SemaphoreType.DMA,
          pltpu.SemaphoreType.DMA,
          pltpu.SemaphoreType.DMA,
          pltpu.SemaphoreType.DMA,
      ],
  )
  def k(gidx_hbm, dst_hbm, tab_hbm, z_hbm, out_hbm,
        gidx_v, dst_v, buf_a, buf_b, acc_sh,
        sem_ga, sem_gb, sem_sa, sem_sb):
    cid = lax.axis_index("c")
    sid = lax.axis_index("s")
    wid = cid * NS + sid

    pltpu.sync_copy(z_hbm, acc_sh.at[pl.ds(sid * ROWS_PT, ROWS_PT)])
    plsc.subcore_barrier()

    def issue_gather(c, buf, sem):
      pltpu.async_copy(tab_hbm.at[gidx_v.at[c]], buf, sem)

    def wait_gather(c, buf, sem):
      pltpu.make_async_copy(tab_hbm.at[gidx_v.at[c]], buf, sem).wait()

    def mul(buf):
      def mul_body(e, carry2):
        for j in range(D // LANES):
          s = pl.ds(j * LANES, LANES)
          buf[e, s] = buf[e, s] * buf[CHUNK + e, s]
        return carry2

      lax.fori_loop(0, CHUNK, mul_body, 0)

    def super_body(sc, carry):
      pltpu.sync_copy(gidx_hbm.at[wid, sc], gidx_v)
      pltpu.sync_copy(dst_hbm.at[wid, sc], dst_v)

      issue_gather(0, buf_a, sem_ga)
      issue_gather(1, buf_b, sem_gb)

      def pair_body(p, carry1):
        c0 = 2 * p
        c1 = c0 + 1
        wait_gather(c0, buf_a, sem_ga)
        mul(buf_a)
        scat_a = pltpu.async_copy(buf_a.at[pl.ds(0, CHUNK)],
                                  acc_sh.at[dst_v.at[c0]], sem_sa, add=True)
        wait_gather(c1, buf_b, sem_gb)
        mul(buf_b)
        scat_b = pltpu.async_copy(buf_b.at[pl.ds(0, CHUNK)],
                                  acc_sh.at[dst_v.at[c1]], sem_sb, add=True)

        @pl.when(p + 1 < NPAIR)
        def _():
          scat_a.wait()
          issue_gather(c0 + 2, buf_a, sem_ga)
          scat_b.wait()
          issue_gather(c1 + 2, buf_b, sem_gb)

        return carry1

      lax.fori_loop(0, NPAIR, pair_body, 0)
      pltpu.make_async_copy(buf_a.at[pl.ds(0, CHUNK)],
                            acc_sh.at[dst_v.at[IDXC - 2]], sem_sa).wait()
      pltpu.make_async_copy(buf_b.at[pl.ds(0, CHUNK)],
                            acc_sh.at[dst_v.at[IDXC - 1]], sem_sb).wait()
      return carry

    lax.fori_loop(0, NSUP, super_body, 0)
    plsc.subcore_barrier()

    r0 = sid * ROWS_PT
    pltpu.sync_copy(acc_sh.at[pl.ds(r0, ROWS_PT)],
                    out_hbm.at[cid, pl.ds(r0, ROWS_PT)])

  return k(gidx4, dst4, tab, zrows)


def _tc_out_matmul(partials, weight, bias2d):
  BM = 1264

  def body(p_ref, w_ref, b_ref, o_ref):
    acc = jnp.dot(p_ref[0] + p_ref[1], w_ref[...],
                  preferred_element_type=jnp.float32)
    o_ref[...] = acc + b_ref[...]

  return pl.pallas_call(
      body,
      grid=(NPAD // BM,),
      in_specs=[
          pl.BlockSpec((NC, BM, D), lambda i: (0, i, 0)),
          pl.BlockSpec((D, D), lambda i: (0, 0)),
          pl.BlockSpec((1, D), lambda i: (0, 0)),
      ],
      out_specs=pl.BlockSpec((BM, D), lambda i: (i, 0)),
      out_shape=jax.ShapeDtypeStruct((NPAD, D), jnp.float32),
  )(partials, weight, bias2d)


def _tc_rel_matmul(re_pad, rel_weight):
  def body(r_ref, w_ref, o_ref):
    o_ref[...] = jnp.dot(r_ref[...], w_ref[...],
                         preferred_element_type=jnp.float32)

  return pl.pallas_call(
      body,
      out_shape=jax.ShapeDtypeStruct((re_pad.shape[0], D), jnp.float32),
  )(re_pad, rel_weight)


def kernel(x, edge_index, edge_type, rel_embed, weight, rel_weight, bias):
  src = edge_index[0]
  dst = edge_index[1]
  npad = EPAD - E
  spread = jnp.arange(npad, dtype=jnp.int32) % N
  src_p = jnp.concatenate([src, spread]).reshape(NW, NSUP, IDXC, CHUNK)
  et_p = jnp.concatenate(
      [edge_type, jnp.full((npad,), 2 * NUM_RELS, jnp.int32)]
  ).reshape(NW, NSUP, IDXC, CHUNK)
  gidx4 = jnp.concatenate([src_p, et_p + N], axis=3)
  dst4 = jnp.concatenate([dst, spread]).reshape(NW, NSUP, IDXC, CHUNK)

  re_ext = jnp.concatenate(
      [rel_embed, jnp.zeros((1, D), rel_embed.dtype)], axis=0)
  tab = jnp.concatenate([x, re_ext], axis=0)
  zrows = jnp.zeros((ROWS_PT, D), jnp.float32)

  partials = _sc_message_accumulate(gidx4, dst4, tab, zrows)
  out = _tc_out_matmul(partials, weight, bias.reshape(1, D))[:N]

  re_pad = jnp.concatenate(
      [re_ext, jnp.zeros((7, D), rel_embed.dtype)], axis=0)
  rel_out = _tc_rel_matmul(re_pad, rel_weight)[:2 * NUM_RELS + 1]
  return (out, rel_out)


# speedup vs baseline: 2.2482x; 1.2194x over previous
import functools

import jax
import jax.numpy as jnp
from jax import lax
from jax.experimental import pallas as pl
from jax.experimental.pallas import tpu as pltpu
from jax.experimental.pallas import tpu_sc as plsc

N = 10000
E = 320000
D = 128
NUM_RELS = 200

NC = 2
NS = 16
NW = NC * NS
CHUNK = 16
GIDX = 2 * CHUNK
NCHUNK = 640
EPW = NCHUNK * CHUNK
IDXC = 64
NSUP = NCHUNK // IDXC
RING = 8
LEAD = RING - 2
GRP = RING
NGRP = IDXC // GRP
EPAD = EPW * NW
NPAD = 10112
ROWS_PT = NPAD // NS
LANES = 16


def _sc_message_accumulate(gidx4, dst4, tab, zrows):
  mesh = plsc.VectorSubcoreMesh(core_axis_name="c", subcore_axis_name="s")

  @functools.partial(
      pl.kernel,
      out_type=jax.ShapeDtypeStruct((NC, NPAD, D), jnp.float32),
      mesh=mesh,
      scratch_types=[
          pltpu.VMEM((IDXC, GIDX), jnp.int32),
          pltpu.VMEM((IDXC, CHUNK), jnp.int32),
          [pltpu.VMEM((GIDX, D), jnp.float32) for _ in range(RING)],
          pltpu.VMEM_SHARED((NPAD, D), jnp.float32),
          [pltpu.SemaphoreType.DMA for _ in range(RING)],
          [pltpu.SemaphoreType.DMA for _ in range(RING)],
      ],
  )
  def k(gidx_hbm, dst_hbm, tab_hbm, z_hbm, out_hbm,
        gidx_v, dst_v, bufs, acc_sh, sem_g, sem_s):
    cid = lax.axis_index("c")
    sid = lax.axis_index("s")
    wid = cid * NS + sid

    pltpu.sync_copy(z_hbm, acc_sh.at[pl.ds(sid * ROWS_PT, ROWS_PT)])
    plsc.subcore_barrier()

    def issue_gather(c, r):
      pltpu.async_copy(tab_hbm.at[gidx_v.at[c]], bufs[r], sem_g[r])

    def wait_gather(c, r):
      pltpu.make_async_copy(tab_hbm.at[gidx_v.at[c]], bufs[r],
                            sem_g[r]).wait()

    def issue_scatter(c, r):
      pltpu.async_copy(bufs[r].at[pl.ds(0, CHUNK)],
                       acc_sh.at[dst_v.at[c]], sem_s[r], add=True)

    def wait_scatter(c, r):
      pltpu.make_async_copy(bufs[r].at[pl.ds(0, CHUNK)],
                            acc_sh.at[dst_v.at[c]], sem_s[r]).wait()

    def mul(r):
      buf = bufs[r]

      def mul_body(e, carry2):
        for j in range(D // LANES):
          s = pl.ds(j * LANES, LANES)
          buf[e, s] = buf[e, s] * buf[CHUNK + e, s]
        return carry2

      lax.fori_loop(0, CHUNK, mul_body, 0)

    def super_body(sc, carry):
      pltpu.sync_copy(gidx_hbm.at[wid, sc], gidx_v)
      pltpu.sync_copy(dst_hbm.at[wid, sc], dst_v)

      for c in range(LEAD):
        issue_gather(c, c % RING)

      def group_body(g, carry1):
        base = g * GRP
        for k in range(GRP):
          c = base + k
          if k >= LEAD:
            wait_scatter(c - LEAD, (k - LEAD) % RING)
          else:
            @pl.when(g >= 1)
            def _():
              wait_scatter(c - LEAD, (k - LEAD + GRP) % RING)
          if k < GRP - LEAD:
            issue_gather(c + LEAD, (k + LEAD) % RING)
          else:
            @pl.when(g + 1 < NGRP)
            def _():
              issue_gather(c + LEAD, (k + LEAD) % RING)
          wait_gather(c, k % RING)
          mul(k % RING)
          issue_scatter(c, k % RING)
        return carry1

      lax.fori_loop(0, NGRP, group_body, 0)
      for i in range(LEAD):
        c = IDXC - LEAD + i
        wait_scatter(c, c % RING)
      return carry

    lax.fori_loop(0, NSUP, super_body, 0)
    plsc.subcore_barrier()

    r0 = sid * ROWS_PT
    pltpu.sync_copy(acc_sh.at[pl.ds(r0, ROWS_PT)],
                    out_hbm.at[cid, pl.ds(r0, ROWS_PT)])

  return k(gidx4, dst4, tab, zrows)


def _tc_out_matmul(partials, weight, bias2d):
  BM = 1264

  def body(p_ref, w_ref, b_ref, o_ref):
    acc = jnp.dot(p_ref[0] + p_ref[1], w_ref[...],
                  preferred_element_type=jnp.float32)
    o_ref[...] = acc + b_ref[...]

  return pl.pallas_call(
      body,
      grid=(NPAD // BM,),
      in_specs=[
          pl.BlockSpec((NC, BM, D), lambda i: (0, i, 0)),
          pl.BlockSpec((D, D), lambda i: (0, 0)),
          pl.BlockSpec((1, D), lambda i: (0, 0)),
      ],
      out_specs=pl.BlockSpec((BM, D), lambda i: (i, 0)),
      out_shape=jax.ShapeDtypeStruct((NPAD, D), jnp.float32),
  )(partials, weight, bias2d)


def _tc_rel_matmul(re_pad, rel_weight):
  def body(r_ref, w_ref, o_ref):
    o_ref[...] = jnp.dot(r_ref[...], w_ref[...],
                         preferred_element_type=jnp.float32)

  return pl.pallas_call(
      body,
      out_shape=jax.ShapeDtypeStruct((re_pad.shape[0], D), jnp.float32),
  )(re_pad, rel_weight)


def kernel(x, edge_index, edge_type, rel_embed, weight, rel_weight, bias):
  src = edge_index[0]
  dst = edge_index[1]
  npad = EPAD - E
  spread = jnp.arange(npad, dtype=jnp.int32) % N
  src_p = jnp.concatenate([src, spread]).reshape(NW, NSUP, IDXC, CHUNK)
  et_p = jnp.concatenate(
      [edge_type, jnp.full((npad,), 2 * NUM_RELS, jnp.int32)]
  ).reshape(NW, NSUP, IDXC, CHUNK)
  gidx4 = jnp.concatenate([src_p, et_p + N], axis=3)
  dst4 = jnp.concatenate([dst, spread]).reshape(NW, NSUP, IDXC, CHUNK)

  re_ext = jnp.concatenate(
      [rel_embed, jnp.zeros((1, D), rel_embed.dtype)], axis=0)
  tab = jnp.concatenate([x, re_ext], axis=0)
  zrows = jnp.zeros((ROWS_PT, D), jnp.float32)

  partials = _sc_message_accumulate(gidx4, dst4, tab, zrows)
  out = _tc_out_matmul(partials, weight, bias.reshape(1, D))[:N]

  re_pad = jnp.concatenate(
      [re_ext, jnp.zeros((7, D), rel_embed.dtype)], axis=0)
  rel_out = _tc_rel_matmul(re_pad, rel_weight)[:2 * NUM_RELS + 1]
  return (out, rel_out)
